# baseline (speedup 1.0000x reference)
import functools

import jax
import jax.numpy as jnp
from jax import lax
from jax.experimental import pallas as pl
from jax.experimental.pallas import tpu as pltpu
from jax.experimental.pallas import tpu_sc as plsc

_NG = 48
_NNGB = 13
_S = 1024
_NB = 128
_DIM = 3
_NBP = 16
_NW = 32
_CHANS = [(2, 8), (8, 8), (8, 8), (8, 8), (8, 1)]

_f32 = jnp.float32
_bf16 = jnp.bfloat16


def _split(x):
    hi = x.astype(_bf16)
    lo = (x - hi.astype(_f32)).astype(_bf16)
    return hi, lo


def _softplus(y):
    return jnp.maximum(y, 0.0) + jnp.log1p(jnp.exp(-jnp.abs(y)))


def _dot3(wh, wl, rh, rl):
    return (
        jnp.dot(wh, rh, preferred_element_type=_f32)
        + jnp.dot(wh, rl, preferred_element_type=_f32)
        + jnp.dot(wl, rh, preferred_element_type=_f32)
    )


@functools.cache
def _make_sc_gather(R):
    pairs = R // _NW
    nrow = _NNGB * _S
    mesh = plsc.VectorSubcoreMesh(core_axis_name="c", subcore_axis_name="s")

    @functools.partial(
        pl.kernel,
        out_type=jax.ShapeDtypeStruct((R * nrow,), _f32),
        mesh=mesh,
        scratch_types=[
            pltpu.VMEM((nrow,), jnp.int32),
            pltpu.VMEM((_S,), _f32),
            pltpu.VMEM((nrow,), _f32),
        ],
        compiler_params=pltpu.CompilerParams(needs_layout_passes=False),
    )
    def gk(x_hbm, nn_hbm, out_hbm, idx_v, xrow_v, orow_v):
        wid = lax.axis_index("s") * 2 + lax.axis_index("c")
        pltpu.sync_copy(nn_hbm, idx_v)

        def pbody(p, carry):
            r = wid * pairs + p
            pltpu.sync_copy(x_hbm.at[pl.ds(r * _S, _S)], xrow_v)

            def tbody(t, c2):
                off = t * 16
                iv = idx_v[pl.ds(off, 16)]
                orow_v[pl.ds(off, 16)] = plsc.load_gather(xrow_v, [iv])
                return c2

            lax.fori_loop(0, nrow // 16, tbody, 0, unroll=8)
            pltpu.sync_copy(orow_v, out_hbm.at[pl.ds(r * nrow, nrow)])
            return carry

        lax.fori_loop(0, pairs, pbody, 0)

    return gk


def _sc_gather(x, nnflat, R):
    return _make_sc_gather(R)(x.reshape(R * _S), nnflat)


def _conv_layer(g, wh, wl, gb, C, O):
    grid = (_NB // _NBP,)

    def body(g_ref, wh_ref, wl_ref, gb_ref, out_ref):
        Wh = wh_ref[...]
        Wl = wl_ref[...]
        Gb = gb_ref[...]

        def conv_b(b, carry):
            rbc = g_ref[b].reshape(C * _NNGB, _S)
            rh, rl = _split(rbc)
            Y = _dot3(Wh, Wl, rh, rl) + Gb
            Sp = _softplus(Y).reshape(O, _NG, _S)
            out_ref[pl.ds(b * O, O), :] = jnp.sum(Sp, axis=1) * (1.0 / _NG)
            return carry

        lax.fori_loop(0, _NBP, conv_b, 0)

    return pl.pallas_call(
        body,
        grid=grid,
        in_specs=[
            pl.BlockSpec((_NBP, C, _NNGB, _S), lambda i: (i, 0, 0, 0)),
            pl.BlockSpec(wh.shape, lambda i: (0, 0)),
            pl.BlockSpec(wl.shape, lambda i: (0, 0)),
            pl.BlockSpec(gb.shape, lambda i: (0, 0)),
        ],
        out_specs=pl.BlockSpec((_NBP * O, _S), lambda i: (i, 0)),
        out_shape=jax.ShapeDtypeStruct((_NB * O, _S), _f32),
        compiler_params=pltpu.CompilerParams(
            dimension_semantics=("arbitrary",),
        ),
    )(g, wh, wl, gb)


def _final_layer(g, gdh, gdl, wph, wpl):
    grid = (_NB // _NBP,)

    def body(g_ref, gdh_ref, gdl_ref, wph_ref, wpl_ref, out_ref):
        T = _dot3(gdh_ref[...], gdl_ref[...], wph_ref[...], wpl_ref[...])
        Th, Tl = _split(T)

        def out_b(b, carry):
            rbc = g_ref[b].reshape(_NNGB, _S)
            rh, rl = _split(rbc)
            Yb = _dot3(Th, Tl, rh, rl)
            out_ref[b] = jnp.sum(Yb.reshape(_NG, _DIM, _S), axis=0) * (1.0 / _NG)
            return carry

        lax.fori_loop(0, _NBP, out_b, 0)

    return pl.pallas_call(
        body,
        grid=grid,
        in_specs=[
            pl.BlockSpec((_NBP, 1, _NNGB, _S), lambda i: (i, 0, 0, 0)),
            pl.BlockSpec(gdh.shape, lambda i: (0, 0)),
            pl.BlockSpec(gdl.shape, lambda i: (0, 0)),
            pl.BlockSpec(wph.shape, lambda i: (0, 0)),
            pl.BlockSpec(wpl.shape, lambda i: (0, 0)),
        ],
        out_specs=pl.BlockSpec((_NBP, _DIM, _S), lambda i: (i, 0, 0)),
        out_shape=jax.ShapeDtypeStruct((_NB, _DIM, _S), _f32),
        compiler_params=pltpu.CompilerParams(
            dimension_semantics=("arbitrary",),
        ),
    )(g, gdh, gdl, wph, wpl)


def kernel(InState, GnnPerms, NNsites, gdiags, Psi0, bias0, Psi1, bias1,
           Psi2, bias2, Psi3, bias3, Psi4, bias4, wtVC):
    Psis = [Psi0, Psi1, Psi2, Psi3, Psi4]
    biases = [bias0, bias1, bias2, bias3, bias4]

    nnflat = NNsites.astype(jnp.int32).reshape(_NNGB * _S)

    w_list, b_list = [], []
    for (C, O), Psi, bias in zip(_CHANS, Psis, biases):
        wrep = jnp.repeat(Psi, _NG, axis=0)
        perm = jnp.tile(GnnPerms, (O, C)).reshape(-1, C, _NNGB)
        GW = jnp.take_along_axis(wrep, perm, axis=2).reshape(
            O * _NG, C * _NNGB
        )
        w_list.append(_split(GW))
        b_list.append(jnp.repeat(bias, _NG, axis=0))

    wt_rep = jnp.tile(wtVC, (_NG, 1))
    perm = jnp.repeat(GnnPerms, _DIM, axis=0)
    wtp = jnp.take_along_axis(wt_rep, perm, axis=1)
    wph, wpl = _split(wtp)
    gdh, gdl = _split(gdiags)

    x = InState.reshape(_NB * _CHANS[0][0], _S)
    for (C, O), (wh, wl), gb in zip(_CHANS, w_list, b_list):
        g = _sc_gather(x, nnflat, _NB * C).reshape(_NB, C, _NNGB, _S)
        x = _conv_layer(g, wh, wl, gb, C, O)
    g = _sc_gather(x, nnflat, _NB).reshape(_NB, 1, _NNGB, _S)
    return _final_layer(g, gdh, gdl, wph, wpl)


# speedup vs baseline: 1.4777x; 1.4777x over previous
import functools

import jax
import jax.numpy as jnp
from jax import lax
from jax.experimental import pallas as pl
from jax.experimental.pallas import tpu as pltpu
from jax.experimental.pallas import tpu_sc as plsc

_NG = 48
_NNGB = 13
_S = 1024
_NB = 128
_DIM = 3
_NBP = 16
_NW = 32
_CHANS = [(2, 8), (8, 8), (8, 8), (8, 8), (8, 1)]

_f32 = jnp.float32
_bf16 = jnp.bfloat16


def _split(x):
    hi = x.astype(_bf16)
    lo = (x - hi.astype(_f32)).astype(_bf16)
    return hi, lo


def _softplus(y):
    return jnp.maximum(y, 0.0) + jnp.log1p(jnp.exp(-jnp.abs(y)))


def _dot3(wh, wl, rh, rl):
    return (
        jnp.dot(wh, rh, preferred_element_type=_f32)
        + jnp.dot(wh, rl, preferred_element_type=_f32)
        + jnp.dot(wl, rh, preferred_element_type=_f32)
    )


@functools.cache
def _make_sc_gather(R):
    pairs = R // _NW
    nrow = _NNGB * _S
    mesh = plsc.VectorSubcoreMesh(core_axis_name="c", subcore_axis_name="s")

    @functools.partial(
        pl.kernel,
        out_type=jax.ShapeDtypeStruct((R * nrow,), _f32),
        mesh=mesh,
        scratch_types=[
            pltpu.VMEM((nrow,), jnp.int32),
            pltpu.VMEM((_S,), _f32),
            pltpu.VMEM((nrow,), _f32),
        ],
        compiler_params=pltpu.CompilerParams(needs_layout_passes=False),
    )
    def gk(x_hbm, nn_hbm, out_hbm, idx_v, xrow_v, orow_v):
        wid = lax.axis_index("s") * 2 + lax.axis_index("c")
        pltpu.sync_copy(nn_hbm, idx_v)

        def pbody(p, carry):
            r = wid * pairs + p
            pltpu.sync_copy(x_hbm.at[pl.ds(r * _S, _S)], xrow_v)

            @plsc.parallel_loop(0, nrow // 16, unroll=8)
            def tbody(t):
                off = t * 16
                iv = idx_v[pl.ds(off, 16)]
                orow_v[pl.ds(off, 16)] = plsc.load_gather(xrow_v, [iv])

            pltpu.sync_copy(orow_v, out_hbm.at[pl.ds(r * nrow, nrow)])
            return carry

        lax.fori_loop(0, pairs, pbody, 0)

    return gk


def _sc_gather(x, nnflat, R):
    return _make_sc_gather(R)(x.reshape(R * _S), nnflat)


def _conv_layer(g, wh, wl, gb, C, O):
    grid = (_NB // _NBP,)

    def body(g_ref, wh_ref, wl_ref, gb_ref, out_ref):
        Wh = wh_ref[...]
        Wl = wl_ref[...]
        Gb = gb_ref[...]

        def conv_b(b, carry):
            rbc = g_ref[b].reshape(C * _NNGB, _S)
            rh, rl = _split(rbc)
            Y = _dot3(Wh, Wl, rh, rl) + Gb
            Sp = _softplus(Y).reshape(O, _NG, _S)
            out_ref[pl.ds(b * O, O), :] = jnp.sum(Sp, axis=1) * (1.0 / _NG)
            return carry

        lax.fori_loop(0, _NBP, conv_b, 0)

    return pl.pallas_call(
        body,
        grid=grid,
        in_specs=[
            pl.BlockSpec((_NBP, C, _NNGB, _S), lambda i: (i, 0, 0, 0)),
            pl.BlockSpec(wh.shape, lambda i: (0, 0)),
            pl.BlockSpec(wl.shape, lambda i: (0, 0)),
            pl.BlockSpec(gb.shape, lambda i: (0, 0)),
        ],
        out_specs=pl.BlockSpec((_NBP * O, _S), lambda i: (i, 0)),
        out_shape=jax.ShapeDtypeStruct((_NB * O, _S), _f32),
        compiler_params=pltpu.CompilerParams(
            dimension_semantics=("arbitrary",),
        ),
    )(g, wh, wl, gb)


def _final_layer(g, gdh, gdl, wph, wpl):
    grid = (_NB // _NBP,)

    def body(g_ref, gdh_ref, gdl_ref, wph_ref, wpl_ref, out_ref):
        T = _dot3(gdh_ref[...], gdl_ref[...], wph_ref[...], wpl_ref[...])
        Th, Tl = _split(T)

        def out_b(b, carry):
            rbc = g_ref[b].reshape(_NNGB, _S)
            rh, rl = _split(rbc)
            Yb = _dot3(Th, Tl, rh, rl)
            out_ref[b] = jnp.sum(Yb.reshape(_NG, _DIM, _S), axis=0) * (1.0 / _NG)
            return carry

        lax.fori_loop(0, _NBP, out_b, 0)

    return pl.pallas_call(
        body,
        grid=grid,
        in_specs=[
            pl.BlockSpec((_NBP, 1, _NNGB, _S), lambda i: (i, 0, 0, 0)),
            pl.BlockSpec(gdh.shape, lambda i: (0, 0)),
            pl.BlockSpec(gdl.shape, lambda i: (0, 0)),
            pl.BlockSpec(wph.shape, lambda i: (0, 0)),
            pl.BlockSpec(wpl.shape, lambda i: (0, 0)),
        ],
        out_specs=pl.BlockSpec((_NBP, _DIM, _S), lambda i: (i, 0, 0)),
        out_shape=jax.ShapeDtypeStruct((_NB, _DIM, _S), _f32),
        compiler_params=pltpu.CompilerParams(
            dimension_semantics=("arbitrary",),
        ),
    )(g, gdh, gdl, wph, wpl)


def kernel(InState, GnnPerms, NNsites, gdiags, Psi0, bias0, Psi1, bias1,
           Psi2, bias2, Psi3, bias3, Psi4, bias4, wtVC):
    Psis = [Psi0, Psi1, Psi2, Psi3, Psi4]
    biases = [bias0, bias1, bias2, bias3, bias4]

    nnflat = NNsites.astype(jnp.int32).reshape(_NNGB * _S)

    w_list, b_list = [], []
    for (C, O), Psi, bias in zip(_CHANS, Psis, biases):
        wrep = jnp.repeat(Psi, _NG, axis=0)
        perm = jnp.tile(GnnPerms, (O, C)).reshape(-1, C, _NNGB)
        GW = jnp.take_along_axis(wrep, perm, axis=2).reshape(
            O * _NG, C * _NNGB
        )
        w_list.append(_split(GW))
        b_list.append(jnp.repeat(bias, _NG, axis=0))

    wt_rep = jnp.tile(wtVC, (_NG, 1))
    perm = jnp.repeat(GnnPerms, _DIM, axis=0)
    wtp = jnp.take_along_axis(wt_rep, perm, axis=1)
    wph, wpl = _split(wtp)
    gdh, gdl = _split(gdiags)

    x = InState.reshape(_NB * _CHANS[0][0], _S)
    for (C, O), (wh, wl), gb in zip(_CHANS, w_list, b_list):
        g = _sc_gather(x, nnflat, _NB * C).reshape(_NB, C, _NNGB, _S)
        x = _conv_layer(g, wh, wl, gb, C, O)
    g = _sc_gather(x, nnflat, _NB).reshape(_NB, 1, _NNGB, _S)
    return _final_layer(g, gdh, gdl, wph, wpl)


# speedup vs baseline: 1.4799x; 1.0015x over previous
import functools

import jax
import jax.numpy as jnp
from jax import lax
from jax.experimental import pallas as pl
from jax.experimental.pallas import tpu as pltpu
from jax.experimental.pallas import tpu_sc as plsc

_NG = 48
_NNGB = 13
_S = 1024
_NB = 128
_DIM = 3
_NBP = 16
_NW = 32
_CHANS = [(2, 8), (8, 8), (8, 8), (8, 8), (8, 1)]

_f32 = jnp.float32
_bf16 = jnp.bfloat16


def _split(x):
    hi = x.astype(_bf16)
    lo = (x - hi.astype(_f32)).astype(_bf16)
    return hi, lo


def _softplus(y):
    return jnp.maximum(y, 0.0) + jnp.log1p(jnp.exp(-jnp.abs(y)))


def _dot3(wh, wl, rh, rl):
    return (
        jnp.dot(wh, rh, preferred_element_type=_f32)
        + jnp.dot(wh, rl, preferred_element_type=_f32)
        + jnp.dot(wl, rh, preferred_element_type=_f32)
    )


@functools.cache
def _make_sc_gather(R):
    pairs = R // _NW
    nrow = _NNGB * _S
    mesh = plsc.VectorSubcoreMesh(core_axis_name="c", subcore_axis_name="s")

    @functools.partial(
        pl.kernel,
        out_type=jax.ShapeDtypeStruct((R * nrow,), _f32),
        mesh=mesh,
        scratch_types=[
            pltpu.VMEM((nrow,), jnp.int32),
            pltpu.VMEM((_S,), _f32),
            pltpu.VMEM((nrow,), _f32),
        ],
        compiler_params=pltpu.CompilerParams(needs_layout_passes=False),
    )
    def gk(x_hbm, nn_hbm, out_hbm, idx_v, xrow_v, orow_v):
        wid = lax.axis_index("s") * 2 + lax.axis_index("c")
        pltpu.sync_copy(nn_hbm, idx_v)

        def pbody(p, carry):
            r = wid * pairs + p
            pltpu.sync_copy(x_hbm.at[pl.ds(r * _S, _S)], xrow_v)

            @plsc.parallel_loop(0, nrow // 16, unroll=16)
            def tbody(t):
                off = t * 16
                iv = idx_v[pl.ds(off, 16)]
                orow_v[pl.ds(off, 16)] = plsc.load_gather(xrow_v, [iv])

            pltpu.sync_copy(orow_v, out_hbm.at[pl.ds(r * nrow, nrow)])
            return carry

        lax.fori_loop(0, pairs, pbody, 0)

    return gk


def _sc_gather(x, nnflat, R):
    return _make_sc_gather(R)(x.reshape(R * _S), nnflat)


def _conv_layer(g, wh, wl, gb, C, O):
    grid = (_NB // _NBP,)

    def body(g_ref, wh_ref, wl_ref, gb_ref, out_ref):
        Wh = wh_ref[...]
        Wl = wl_ref[...]
        Gb = gb_ref[...]

        def conv_b(b, carry):
            rbc = g_ref[b].reshape(C * _NNGB, _S)
            rh, rl = _split(rbc)
            Y = _dot3(Wh, Wl, rh, rl) + Gb
            Sp = _softplus(Y).reshape(O, _NG, _S)
            out_ref[pl.ds(b * O, O), :] = jnp.sum(Sp, axis=1) * (1.0 / _NG)
            return carry

        lax.fori_loop(0, _NBP, conv_b, 0)

    return pl.pallas_call(
        body,
        grid=grid,
        in_specs=[
            pl.BlockSpec((_NBP, C, _NNGB, _S), lambda i: (i, 0, 0, 0)),
            pl.BlockSpec(wh.shape, lambda i: (0, 0)),
            pl.BlockSpec(wl.shape, lambda i: (0, 0)),
            pl.BlockSpec(gb.shape, lambda i: (0, 0)),
        ],
        out_specs=pl.BlockSpec((_NBP * O, _S), lambda i: (i, 0)),
        out_shape=jax.ShapeDtypeStruct((_NB * O, _S), _f32),
        compiler_params=pltpu.CompilerParams(
            dimension_semantics=("arbitrary",),
        ),
    )(g, wh, wl, gb)


def _final_layer(g, gdh, gdl, wph, wpl):
    grid = (_NB // _NBP,)

    def body(g_ref, gdh_ref, gdl_ref, wph_ref, wpl_ref, out_ref):
        T = _dot3(gdh_ref[...], gdl_ref[...], wph_ref[...], wpl_ref[...])
        Th, Tl = _split(T)

        def out_b(b, carry):
            rbc = g_ref[b].reshape(_NNGB, _S)
            rh, rl = _split(rbc)
            Yb = _dot3(Th, Tl, rh, rl)
            out_ref[b] = jnp.sum(Yb.reshape(_NG, _DIM, _S), axis=0) * (1.0 / _NG)
            return carry

        lax.fori_loop(0, _NBP, out_b, 0)

    return pl.pallas_call(
        body,
        grid=grid,
        in_specs=[
            pl.BlockSpec((_NBP, 1, _NNGB, _S), lambda i: (i, 0, 0, 0)),
            pl.BlockSpec(gdh.shape, lambda i: (0, 0)),
            pl.BlockSpec(gdl.shape, lambda i: (0, 0)),
            pl.BlockSpec(wph.shape, lambda i: (0, 0)),
            pl.BlockSpec(wpl.shape, lambda i: (0, 0)),
        ],
        out_specs=pl.BlockSpec((_NBP, _DIM, _S), lambda i: (i, 0, 0)),
        out_shape=jax.ShapeDtypeStruct((_NB, _DIM, _S), _f32),
        compiler_params=pltpu.CompilerParams(
            dimension_semantics=("arbitrary",),
        ),
    )(g, gdh, gdl, wph, wpl)


def kernel(InState, GnnPerms, NNsites, gdiags, Psi0, bias0, Psi1, bias1,
           Psi2, bias2, Psi3, bias3, Psi4, bias4, wtVC):
    Psis = [Psi0, Psi1, Psi2, Psi3, Psi4]
    biases = [bias0, bias1, bias2, bias3, bias4]

    nnflat = NNsites.astype(jnp.int32).reshape(_NNGB * _S)

    w_list, b_list = [], []
    for (C, O), Psi, bias in zip(_CHANS, Psis, biases):
        wrep = jnp.repeat(Psi, _NG, axis=0)
        perm = jnp.tile(GnnPerms, (O, C)).reshape(-1, C, _NNGB)
        GW = jnp.take_along_axis(wrep, perm, axis=2).reshape(
            O * _NG, C * _NNGB
        )
        w_list.append(_split(GW))
        b_list.append(jnp.repeat(bias, _NG, axis=0))

    wt_rep = jnp.tile(wtVC, (_NG, 1))
    perm = jnp.repeat(GnnPerms, _DIM, axis=0)
    wtp = jnp.take_along_axis(wt_rep, perm, axis=1)
    wph, wpl = _split(wtp)
    gdh, gdl = _split(gdiags)

    x = InState.reshape(_NB * _CHANS[0][0], _S)
    for (C, O), (wh, wl), gb in zip(_CHANS, w_list, b_list):
        g = _sc_gather(x, nnflat, _NB * C).reshape(_NB, C, _NNGB, _S)
        x = _conv_layer(g, wh, wl, gb, C, O)
    g = _sc_gather(x, nnflat, _NB).reshape(_NB, 1, _NNGB, _S)
    return _final_layer(g, gdh, gdl, wph, wpl)


# speedup vs baseline: 1.5581x; 1.0528x over previous
import functools

import jax
import jax.numpy as jnp
from jax import lax
from jax.experimental import pallas as pl
from jax.experimental.pallas import tpu as pltpu
from jax.experimental.pallas import tpu_sc as plsc

_NG = 48
_NNGB = 13
_S = 1024
_NB = 128
_DIM = 3
_NBP = 16
_NW = 32
_CHANS = [(2, 8), (8, 8), (8, 8), (8, 8), (8, 1)]

_f32 = jnp.float32
_bf16 = jnp.bfloat16


def _split(x):
    hi = x.astype(_bf16)
    lo = (x - hi.astype(_f32)).astype(_bf16)
    return hi, lo


def _softplus(y):
    return jnp.maximum(y, 0.0) + jnp.log1p(jnp.exp(-jnp.abs(y)))


def _dot3(wh, wl, rh, rl):
    return (
        jnp.dot(wh, rh, preferred_element_type=_f32)
        + jnp.dot(wh, rl, preferred_element_type=_f32)
        + jnp.dot(wl, rh, preferred_element_type=_f32)
    )


@functools.cache
def _make_sc_gather(R):
    pairs = R // _NW
    nrow = _NNGB * _S
    mesh = plsc.VectorSubcoreMesh(core_axis_name="c", subcore_axis_name="s")

    @functools.partial(
        pl.kernel,
        out_type=jax.ShapeDtypeStruct((R * nrow,), _f32),
        mesh=mesh,
        scratch_types=[
            pltpu.VMEM((nrow,), jnp.int32),
            pltpu.VMEM((pairs * _S,), _f32),
            pltpu.VMEM((2, nrow), _f32),
            pltpu.SemaphoreType.DMA,
            pltpu.SemaphoreType.DMA,
        ],
        compiler_params=pltpu.CompilerParams(needs_layout_passes=False),
    )
    def gk(x_hbm, nn_hbm, out_hbm, idx_v, xall_v, orow_v, sem0, sem1):
        wid = lax.axis_index("s") * 2 + lax.axis_index("c")
        pltpu.sync_copy(nn_hbm, idx_v)
        pltpu.sync_copy(x_hbm.at[pl.ds(wid * pairs * _S, pairs * _S)], xall_v)
        sems = (sem0, sem1)

        def p2body(p2, carry):
            for u in range(2):
                p = p2 * 2 + u
                @pl.when(p2 > 0)
                def _drain(u=u):
                    pltpu.make_async_copy(
                        orow_v.at[u], out_hbm.at[pl.ds(0, nrow)], sems[u]
                    ).wait()

                @plsc.parallel_loop(0, nrow // 16, unroll=16)
                def tbody(t, u=u, p=p):
                    off = t * 16
                    iv = idx_v[pl.ds(off, 16)] + p * _S
                    orow_v[u, pl.ds(off, 16)] = plsc.load_gather(xall_v, [iv])

                pltpu.async_copy(
                    orow_v.at[u],
                    out_hbm.at[pl.ds((wid * pairs + p) * nrow, nrow)],
                    sems[u],
                )
            return carry

        lax.fori_loop(0, pairs // 2, p2body, 0)
        for u in range(2):
            pltpu.make_async_copy(
                orow_v.at[u], out_hbm.at[pl.ds(0, nrow)], sems[u]
            ).wait()

    return gk


def _sc_gather(x, nnflat, R):
    return _make_sc_gather(R)(x.reshape(R * _S), nnflat)


def _conv_layer(g, wh, wl, gb, C, O):
    grid = (_NB // _NBP,)

    def body(g_ref, wh_ref, wl_ref, gb_ref, out_ref):
        Wh = wh_ref[...]
        Wl = wl_ref[...]
        Gb = gb_ref[...]

        def conv_b(b, carry):
            rbc = g_ref[b].reshape(C * _NNGB, _S)
            rh, rl = _split(rbc)
            Y = _dot3(Wh, Wl, rh, rl) + Gb
            Sp = _softplus(Y).reshape(O, _NG, _S)
            out_ref[pl.ds(b * O, O), :] = jnp.sum(Sp, axis=1) * (1.0 / _NG)
            return carry

        lax.fori_loop(0, _NBP, conv_b, 0)

    return pl.pallas_call(
        body,
        grid=grid,
        in_specs=[
            pl.BlockSpec((_NBP, C, _NNGB, _S), lambda i: (i, 0, 0, 0)),
            pl.BlockSpec(wh.shape, lambda i: (0, 0)),
            pl.BlockSpec(wl.shape, lambda i: (0, 0)),
            pl.BlockSpec(gb.shape, lambda i: (0, 0)),
        ],
        out_specs=pl.BlockSpec((_NBP * O, _S), lambda i: (i, 0)),
        out_shape=jax.ShapeDtypeStruct((_NB * O, _S), _f32),
        compiler_params=pltpu.CompilerParams(
            dimension_semantics=("arbitrary",),
        ),
    )(g, wh, wl, gb)


def _final_layer(g, gdh, gdl, wph, wpl):
    grid = (_NB // _NBP,)

    def body(g_ref, gdh_ref, gdl_ref, wph_ref, wpl_ref, out_ref):
        T = _dot3(gdh_ref[...], gdl_ref[...], wph_ref[...], wpl_ref[...])
        Th, Tl = _split(T)

        def out_b(b, carry):
            rbc = g_ref[b].reshape(_NNGB, _S)
            rh, rl = _split(rbc)
            Yb = _dot3(Th, Tl, rh, rl)
            out_ref[b] = jnp.sum(Yb.reshape(_NG, _DIM, _S), axis=0) * (1.0 / _NG)
            return carry

        lax.fori_loop(0, _NBP, out_b, 0)

    return pl.pallas_call(
        body,
        grid=grid,
        in_specs=[
            pl.BlockSpec((_NBP, 1, _NNGB, _S), lambda i: (i, 0, 0, 0)),
            pl.BlockSpec(gdh.shape, lambda i: (0, 0)),
            pl.BlockSpec(gdl.shape, lambda i: (0, 0)),
            pl.BlockSpec(wph.shape, lambda i: (0, 0)),
            pl.BlockSpec(wpl.shape, lambda i: (0, 0)),
        ],
        out_specs=pl.BlockSpec((_NBP, _DIM, _S), lambda i: (i, 0, 0)),
        out_shape=jax.ShapeDtypeStruct((_NB, _DIM, _S), _f32),
        compiler_params=pltpu.CompilerParams(
            dimension_semantics=("arbitrary",),
        ),
    )(g, gdh, gdl, wph, wpl)


def kernel(InState, GnnPerms, NNsites, gdiags, Psi0, bias0, Psi1, bias1,
           Psi2, bias2, Psi3, bias3, Psi4, bias4, wtVC):
    Psis = [Psi0, Psi1, Psi2, Psi3, Psi4]
    biases = [bias0, bias1, bias2, bias3, bias4]

    nnflat = NNsites.astype(jnp.int32).reshape(_NNGB * _S)

    w_list, b_list = [], []
    for (C, O), Psi, bias in zip(_CHANS, Psis, biases):
        wrep = jnp.repeat(Psi, _NG, axis=0)
        perm = jnp.tile(GnnPerms, (O, C)).reshape(-1, C, _NNGB)
        GW = jnp.take_along_axis(wrep, perm, axis=2).reshape(
            O * _NG, C * _NNGB
        )
        w_list.append(_split(GW))
        b_list.append(jnp.repeat(bias, _NG, axis=0))

    wt_rep = jnp.tile(wtVC, (_NG, 1))
    perm = jnp.repeat(GnnPerms, _DIM, axis=0)
    wtp = jnp.take_along_axis(wt_rep, perm, axis=1)
    wph, wpl = _split(wtp)
    gdh, gdl = _split(gdiags)

    x = InState.reshape(_NB * _CHANS[0][0], _S)
    for (C, O), (wh, wl), gb in zip(_CHANS, w_list, b_list):
        g = _sc_gather(x, nnflat, _NB * C).reshape(_NB, C, _NNGB, _S)
        x = _conv_layer(g, wh, wl, gb, C, O)
    g = _sc_gather(x, nnflat, _NB).reshape(_NB, 1, _NNGB, _S)
    return _final_layer(g, gdh, gdl, wph, wpl)


# speedup vs baseline: 1.7250x; 1.1071x over previous
import functools

import jax
import jax.numpy as jnp
from jax import lax
from jax.experimental import pallas as pl
from jax.experimental.pallas import tpu as pltpu
from jax.experimental.pallas import tpu_sc as plsc

_NG = 48
_NNGB = 13
_S = 1024
_NB = 128
_DIM = 3
_NBP = 16
_NW = 32
_CHANS = [(2, 8), (8, 8), (8, 8), (8, 8), (8, 1)]

_f32 = jnp.float32
_bf16 = jnp.bfloat16


def _split(x):
    hi = x.astype(_bf16)
    lo = (x - hi.astype(_f32)).astype(_bf16)
    return hi, lo


def _softplus(y):
    return jnp.maximum(y, 0.0) + jnp.log1p(jnp.exp(-jnp.abs(y)))


def _dot3(wh, wl, rh, rl):
    return (
        jnp.dot(wh, rh, preferred_element_type=_f32)
        + jnp.dot(wh, rl, preferred_element_type=_f32)
        + jnp.dot(wl, rh, preferred_element_type=_f32)
    )


@functools.cache
def _make_sc_gather(R):
    pairs = R // _NW
    nrow = _NNGB * _S
    mesh = plsc.VectorSubcoreMesh(core_axis_name="c", subcore_axis_name="s")

    @functools.partial(
        pl.kernel,
        out_type=jax.ShapeDtypeStruct((R * nrow,), _f32),
        mesh=mesh,
        scratch_types=[
            pltpu.VMEM((nrow,), jnp.int32),
            pltpu.VMEM((pairs * _S,), _f32),
            pltpu.VMEM((2, nrow), _f32),
            pltpu.SemaphoreType.DMA,
            pltpu.SemaphoreType.DMA,
        ],
        compiler_params=pltpu.CompilerParams(needs_layout_passes=False),
    )
    def gk(x_hbm, nn_hbm, out_hbm, idx_v, xall_v, orow_v, sem0, sem1):
        wid = lax.axis_index("s") * 2 + lax.axis_index("c")
        pltpu.sync_copy(nn_hbm, idx_v)
        pltpu.sync_copy(x_hbm.at[pl.ds(wid * pairs * _S, pairs * _S)], xall_v)
        sems = (sem0, sem1)

        def p2body(p2, carry):
            for u in range(2):
                p = p2 * 2 + u
                @pl.when(p2 > 0)
                def _drain(u=u):
                    pltpu.make_async_copy(
                        orow_v.at[u], out_hbm.at[pl.ds(0, nrow)], sems[u]
                    ).wait()

                @plsc.parallel_loop(0, nrow // 16, unroll=16)
                def tbody(t, u=u, p=p):
                    off = t * 16
                    iv = idx_v[pl.ds(off, 16)] + p * _S
                    orow_v[u, pl.ds(off, 16)] = plsc.load_gather(xall_v, [iv])

                pltpu.async_copy(
                    orow_v.at[u],
                    out_hbm.at[pl.ds((wid * pairs + p) * nrow, nrow)],
                    sems[u],
                )
            return carry

        lax.fori_loop(0, pairs // 2, p2body, 0)
        for u in range(2):
            pltpu.make_async_copy(
                orow_v.at[u], out_hbm.at[pl.ds(0, nrow)], sems[u]
            ).wait()

    return gk


def _sc_gather(x, nnflat, R):
    return _make_sc_gather(R)(x.reshape(R * _S), nnflat)


def _conv_layer(g, wh, wl, gb, C, O):
    grid = (_NB // _NBP,)

    def body(g_ref, whh_ref, wl_ref, gb_ref, out_ref):
        Whh = whh_ref[...]
        Wl = wl_ref[...]
        Gb = gb_ref[...]

        def conv_b(b, carry):
            rbc = g_ref[b].reshape(C * _NNGB, _S)
            rh, rl = _split(rbc)
            r2 = jnp.concatenate([rh, rl], axis=0)
            Y = (
                jnp.dot(Whh, r2, preferred_element_type=_f32)
                + jnp.dot(Wl, rh, preferred_element_type=_f32)
                + Gb
            )
            Sp = _softplus(Y).reshape(O, _NG, _S)
            out_ref[pl.ds(b * O, O), :] = jnp.sum(Sp, axis=1) * (1.0 / _NG)
            return carry

        lax.fori_loop(0, _NBP, conv_b, 0)

    return pl.pallas_call(
        body,
        grid=grid,
        in_specs=[
            pl.BlockSpec((_NBP, C, _NNGB, _S), lambda i: (i, 0, 0, 0)),
            pl.BlockSpec(wh.shape, lambda i: (0, 0)),
            pl.BlockSpec(wl.shape, lambda i: (0, 0)),
            pl.BlockSpec(gb.shape, lambda i: (0, 0)),
        ],
        out_specs=pl.BlockSpec((_NBP * O, _S), lambda i: (i, 0)),
        out_shape=jax.ShapeDtypeStruct((_NB * O, _S), _f32),
        compiler_params=pltpu.CompilerParams(
            dimension_semantics=("arbitrary",),
        ),
    )(g, wh, wl, gb)


def _final_layer(g, gdh, gdl, wph, wpl):
    grid = (_NB // _NBP,)

    def body(g_ref, gdh_ref, gdl_ref, wph_ref, wpl_ref, out_ref):
        T = _dot3(gdh_ref[...], gdl_ref[...], wph_ref[...], wpl_ref[...])
        Th, Tl = _split(T)

        def out_b(b, carry):
            rbc = g_ref[b].reshape(_NNGB, _S)
            rh, rl = _split(rbc)
            Yb = _dot3(Th, Tl, rh, rl)
            out_ref[b] = jnp.sum(Yb.reshape(_NG, _DIM, _S), axis=0) * (1.0 / _NG)
            return carry

        lax.fori_loop(0, _NBP, out_b, 0)

    return pl.pallas_call(
        body,
        grid=grid,
        in_specs=[
            pl.BlockSpec((_NBP, 1, _NNGB, _S), lambda i: (i, 0, 0, 0)),
            pl.BlockSpec(gdh.shape, lambda i: (0, 0)),
            pl.BlockSpec(gdl.shape, lambda i: (0, 0)),
            pl.BlockSpec(wph.shape, lambda i: (0, 0)),
            pl.BlockSpec(wpl.shape, lambda i: (0, 0)),
        ],
        out_specs=pl.BlockSpec((_NBP, _DIM, _S), lambda i: (i, 0, 0)),
        out_shape=jax.ShapeDtypeStruct((_NB, _DIM, _S), _f32),
        compiler_params=pltpu.CompilerParams(
            dimension_semantics=("arbitrary",),
        ),
    )(g, gdh, gdl, wph, wpl)


def kernel(InState, GnnPerms, NNsites, gdiags, Psi0, bias0, Psi1, bias1,
           Psi2, bias2, Psi3, bias3, Psi4, bias4, wtVC):
    Psis = [Psi0, Psi1, Psi2, Psi3, Psi4]
    biases = [bias0, bias1, bias2, bias3, bias4]

    nnflat = NNsites.astype(jnp.int32).reshape(_NNGB * _S)

    w_list, b_list = [], []
    for (C, O), Psi, bias in zip(_CHANS, Psis, biases):
        wrep = jnp.repeat(Psi, _NG, axis=0)
        perm = jnp.tile(GnnPerms, (O, C)).reshape(-1, C, _NNGB)
        GW = jnp.take_along_axis(wrep, perm, axis=2).reshape(
            O * _NG, C * _NNGB
        )
        wh, wl = _split(GW)
        w_list.append((jnp.concatenate([wh, wh], axis=1), wl))
        b_list.append(jnp.repeat(bias, _NG, axis=0))

    wt_rep = jnp.tile(wtVC, (_NG, 1))
    perm = jnp.repeat(GnnPerms, _DIM, axis=0)
    wtp = jnp.take_along_axis(wt_rep, perm, axis=1)
    wph, wpl = _split(wtp)
    gdh, gdl = _split(gdiags)

    x = InState.reshape(_NB * _CHANS[0][0], _S)
    for (C, O), (wh, wl), gb in zip(_CHANS, w_list, b_list):
        g = _sc_gather(x, nnflat, _NB * C).reshape(_NB, C, _NNGB, _S)
        x = _conv_layer(g, wh, wl, gb, C, O)
    g = _sc_gather(x, nnflat, _NB).reshape(_NB, 1, _NNGB, _S)
    return _final_layer(g, gdh, gdl, wph, wpl)


# speedup vs baseline: 1.7261x; 1.0007x over previous
import functools

import jax
import jax.numpy as jnp
from jax import lax
from jax.experimental import pallas as pl
from jax.experimental.pallas import tpu as pltpu
from jax.experimental.pallas import tpu_sc as plsc

_NG = 48
_NNGB = 13
_S = 1024
_NB = 128
_DIM = 3
_NBP = 16
_NW = 32
_CHANS = [(2, 8), (8, 8), (8, 8), (8, 8), (8, 1)]

_f32 = jnp.float32
_bf16 = jnp.bfloat16


def _split(x):
    hi = x.astype(_bf16)
    lo = (x - hi.astype(_f32)).astype(_bf16)
    return hi, lo


def _softplus(y):
    return jnp.maximum(y, 0.0) + jnp.log1p(jnp.exp(-jnp.abs(y)))


def _dot3(wh, wl, rh, rl):
    return (
        jnp.dot(wh, rh, preferred_element_type=_f32)
        + jnp.dot(wh, rl, preferred_element_type=_f32)
        + jnp.dot(wl, rh, preferred_element_type=_f32)
    )


@functools.cache
def _make_sc_gather(R):
    pairs = R // _NW
    nrow = _NNGB * _S
    mesh = plsc.VectorSubcoreMesh(core_axis_name="c", subcore_axis_name="s")

    @functools.partial(
        pl.kernel,
        out_type=jax.ShapeDtypeStruct((R * nrow,), _f32),
        mesh=mesh,
        scratch_types=[
            pltpu.VMEM((nrow,), jnp.int32),
            pltpu.VMEM((pairs * _S,), _f32),
            pltpu.VMEM((2, nrow), _f32),
            pltpu.SemaphoreType.DMA,
            pltpu.SemaphoreType.DMA,
        ],
        compiler_params=pltpu.CompilerParams(needs_layout_passes=False),
    )
    def gk(x_hbm, nn_hbm, out_hbm, idx_v, xall_v, orow_v, sem0, sem1):
        wid = lax.axis_index("s") * 2 + lax.axis_index("c")
        pltpu.sync_copy(nn_hbm, idx_v)
        pltpu.sync_copy(x_hbm.at[pl.ds(wid * pairs * _S, pairs * _S)], xall_v)
        sems = (sem0, sem1)

        def p2body(p2, carry):
            for u in range(2):
                p = p2 * 2 + u
                @pl.when(p2 > 0)
                def _drain(u=u):
                    pltpu.make_async_copy(
                        orow_v.at[u], out_hbm.at[pl.ds(0, nrow)], sems[u]
                    ).wait()

                @plsc.parallel_loop(0, nrow // 16, unroll=16)
                def tbody(t, u=u, p=p):
                    off = t * 16
                    iv = idx_v[pl.ds(off, 16)] + p * _S
                    orow_v[u, pl.ds(off, 16)] = plsc.load_gather(xall_v, [iv])

                pltpu.async_copy(
                    orow_v.at[u],
                    out_hbm.at[pl.ds((wid * pairs + p) * nrow, nrow)],
                    sems[u],
                )
            return carry

        lax.fori_loop(0, pairs // 2, p2body, 0)
        for u in range(2):
            pltpu.make_async_copy(
                orow_v.at[u], out_hbm.at[pl.ds(0, nrow)], sems[u]
            ).wait()

    return gk


def _sc_gather(x, nnflat, R):
    return _make_sc_gather(R)(x.reshape(R * _S), nnflat)


_NBPC = 8


def _conv_layer(g, whh, wl, gb, C, O):
    grid = (_NB // _NBPC,)
    K = C * _NNGB
    NS = _NBPC * _S

    def body(g_ref, whh_ref, wl_ref, gb_ref, out_ref, r2_ref):
        for b in range(_NBPC):
            rbc = g_ref[b].reshape(K, _S)
            rh, rl = _split(rbc)
            r2_ref[:K, b * _S : (b + 1) * _S] = rh
            r2_ref[K:, b * _S : (b + 1) * _S] = rl
        R2 = r2_ref[...]
        Y = (
            jnp.dot(whh_ref[...], R2, preferred_element_type=_f32)
            + jnp.dot(wl_ref[...], R2[:K], preferred_element_type=_f32)
            + gb_ref[...]
        )
        Xp = jnp.sum(_softplus(Y).reshape(O, _NG, NS), axis=1) * (1.0 / _NG)
        for b in range(_NBPC):
            out_ref[b * O : (b + 1) * O, :] = Xp[:, b * _S : (b + 1) * _S]

    return pl.pallas_call(
        body,
        grid=grid,
        in_specs=[
            pl.BlockSpec((_NBPC, C, _NNGB, _S), lambda i: (i, 0, 0, 0)),
            pl.BlockSpec(whh.shape, lambda i: (0, 0)),
            pl.BlockSpec(wl.shape, lambda i: (0, 0)),
            pl.BlockSpec(gb.shape, lambda i: (0, 0)),
        ],
        out_specs=pl.BlockSpec((_NBPC * O, _S), lambda i: (i, 0)),
        out_shape=jax.ShapeDtypeStruct((_NB * O, _S), _f32),
        scratch_shapes=[pltpu.VMEM((2 * K, NS), _bf16)],
        compiler_params=pltpu.CompilerParams(
            dimension_semantics=("arbitrary",),
        ),
    )(g, whh, wl, gb)


def _final_layer(g, gdh, gdl, wph, wpl):
    grid = (_NB // _NBP,)

    def body(g_ref, gdh_ref, gdl_ref, wph_ref, wpl_ref, out_ref):
        T = _dot3(gdh_ref[...], gdl_ref[...], wph_ref[...], wpl_ref[...])
        Th, Tl = _split(T)

        def out_b(b, carry):
            rbc = g_ref[b].reshape(_NNGB, _S)
            rh, rl = _split(rbc)
            Yb = _dot3(Th, Tl, rh, rl)
            out_ref[b] = jnp.sum(Yb.reshape(_NG, _DIM, _S), axis=0) * (1.0 / _NG)
            return carry

        lax.fori_loop(0, _NBP, out_b, 0)

    return pl.pallas_call(
        body,
        grid=grid,
        in_specs=[
            pl.BlockSpec((_NBP, 1, _NNGB, _S), lambda i: (i, 0, 0, 0)),
            pl.BlockSpec(gdh.shape, lambda i: (0, 0)),
            pl.BlockSpec(gdl.shape, lambda i: (0, 0)),
            pl.BlockSpec(wph.shape, lambda i: (0, 0)),
            pl.BlockSpec(wpl.shape, lambda i: (0, 0)),
        ],
        out_specs=pl.BlockSpec((_NBP, _DIM, _S), lambda i: (i, 0, 0)),
        out_shape=jax.ShapeDtypeStruct((_NB, _DIM, _S), _f32),
        compiler_params=pltpu.CompilerParams(
            dimension_semantics=("arbitrary",),
        ),
    )(g, gdh, gdl, wph, wpl)


def kernel(InState, GnnPerms, NNsites, gdiags, Psi0, bias0, Psi1, bias1,
           Psi2, bias2, Psi3, bias3, Psi4, bias4, wtVC):
    Psis = [Psi0, Psi1, Psi2, Psi3, Psi4]
    biases = [bias0, bias1, bias2, bias3, bias4]

    nnflat = NNsites.astype(jnp.int32).reshape(_NNGB * _S)

    w_list, b_list = [], []
    for (C, O), Psi, bias in zip(_CHANS, Psis, biases):
        wrep = jnp.repeat(Psi, _NG, axis=0)
        perm = jnp.tile(GnnPerms, (O, C)).reshape(-1, C, _NNGB)
        GW = jnp.take_along_axis(wrep, perm, axis=2).reshape(
            O * _NG, C * _NNGB
        )
        wh, wl = _split(GW)
        w_list.append((jnp.concatenate([wh, wh], axis=1), wl))
        b_list.append(jnp.repeat(bias, _NG, axis=0))

    wt_rep = jnp.tile(wtVC, (_NG, 1))
    perm = jnp.repeat(GnnPerms, _DIM, axis=0)
    wtp = jnp.take_along_axis(wt_rep, perm, axis=1)
    wph, wpl = _split(wtp)
    gdh, gdl = _split(gdiags)

    x = InState.reshape(_NB * _CHANS[0][0], _S)
    for (C, O), (wh, wl), gb in zip(_CHANS, w_list, b_list):
        g = _sc_gather(x, nnflat, _NB * C).reshape(_NB, C, _NNGB, _S)
        x = _conv_layer(g, wh, wl, gb, C, O)
    g = _sc_gather(x, nnflat, _NB).reshape(_NB, 1, _NNGB, _S)
    return _final_layer(g, gdh, gdl, wph, wpl)
